# baseline (speedup 1.0000x reference)
import functools

import jax
import jax.numpy as jnp
from jax import lax
from jax.experimental import pallas as pl
from jax.experimental.pallas import tpu as pltpu
from jax.experimental.pallas import tpu_sc as plsc

_BATCH = 16384
_EMB = 16
_NC = 2
_NS = 16
_NW = _NC * _NS
_BPW = _BATCH // _NW
_EPS = 1e-5

_mesh = plsc.VectorSubcoreMesh(core_axis_name="c", subcore_axis_name="s")


@functools.partial(
    pl.kernel,
    out_type=jax.ShapeDtypeStruct((_BATCH, _EMB), jnp.float32),
    mesh=_mesh,
    scratch_types=[
        pltpu.VMEM((_BPW,), jnp.int32),
        pltpu.VMEM((_BPW // 2,), jnp.int32),
        pltpu.VMEM((_BPW // 2, 128), jnp.float32),
        pltpu.VMEM((_BPW // 2, _EMB), jnp.float32),
        pltpu.SemaphoreType.DMA,
    ],
)
def _sc_gather(idx_hbm, table_hbm, out_hbm, idx_v, tidx_v, tiles_v, rows_v,
               sem):
    wid = lax.axis_index("s") * _NC + lax.axis_index("c")
    base = wid * _BPW
    half = _BPW // 2
    pltpu.sync_copy(idx_hbm.at[pl.ds(base, _BPW)], idx_v)

    for h in range(2):
        h0 = h * half

        def _mk_tidx(i, _):
            v = idx_v[pl.ds(h0 + i * 16, 16)]
            tidx_v[pl.ds(i * 16, 16)] = lax.shift_right_logical(v, 3)
            return _

        lax.fori_loop(0, half // 16, _mk_tidx, 0, unroll=4)
        pltpu.async_copy(table_hbm.at[tidx_v], tiles_v, sem).wait()

        def _extract(i, _):
            k0 = i * 16
            vec = idx_v[pl.ds(h0 + k0, 16)]
            off = lax.bitwise_and(vec, 7) * 16
            for d in range(16):
                rows_v[k0 + d, :] = tiles_v[k0 + d, pl.ds(off[d], 16)]
            return _

        lax.fori_loop(0, half // 16, _extract, 0)
        pltpu.sync_copy(rows_v, out_hbm.at[pl.ds(base + h0, half)])


def _tc_body(uf_ref, ages_ref, w16t_ref, wage_ref, bias_ref, gb_ref, out_ref):
    a = ages_ref[...]
    n = jnp.float32(_BATCH)
    mean = jnp.sum(a) / n
    d = a - mean
    var = jnp.sum(d * d) / n
    gamma = gb_ref[0, 0]
    beta = gb_ref[0, 1]
    an = d * lax.rsqrt(var + _EPS) * gamma + beta
    an = jnp.maximum(an, 0.0)
    uf = jnp.maximum(uf_ref[...], 0.0)
    out_ref[...] = (
        jnp.dot(uf, w16t_ref[...], preferred_element_type=jnp.float32)
        + an * wage_ref[...]
        + bias_ref[...]
    )


def kernel(user_ids, ages, emb_table, bn_gamma, bn_beta, W, b):
    out_dim = W.shape[0]
    w_pad = jnp.zeros((_EMB, W.shape[1]), W.dtype).at[:out_dim].set(W)
    w16t = w_pad[:, :_EMB].T
    wage = w_pad[:, _EMB].reshape(1, _EMB)
    bias = jnp.zeros((1, _EMB), b.dtype).at[0, :out_dim].set(b)
    gb = jnp.stack([bn_gamma[0], bn_beta[0]]).reshape(1, 2)

    table128 = emb_table.reshape(emb_table.shape[0] // 8, 8 * _EMB)
    uf = _sc_gather(user_ids, table128)

    out_pad = pl.pallas_call(
        _tc_body,
        out_shape=jax.ShapeDtypeStruct((_BATCH, _EMB), jnp.float32),
    )(uf, ages.reshape(_BATCH, 1), w16t, wage, bias, gb)
    return out_pad[:, :out_dim]


# speedup vs baseline: 5.9129x; 5.9129x over previous
import functools

import jax
import jax.numpy as jnp
from jax import lax
from jax.experimental import pallas as pl
from jax.experimental.pallas import tpu as pltpu
from jax.experimental.pallas import tpu_sc as plsc

_BATCH = 16384
_EMB = 16
_NC = 2
_NS = 16
_NW = _NC * _NS
_BPW = _BATCH // _NW
_EPS = 1e-5
_B = 16
_NROUND = _BPW // _B

_mesh = plsc.VectorSubcoreMesh(core_axis_name="c", subcore_axis_name="s")


@functools.partial(
    pl.kernel,
    out_type=jax.ShapeDtypeStruct((_EMB, _BATCH), jnp.float32),
    mesh=_mesh,
    scratch_types=[
        pltpu.VMEM((_BPW,), jnp.int32),
        pltpu.VMEM((2 * _B, _EMB, 128), jnp.float32),
        pltpu.VMEM((_EMB, _BPW), jnp.float32),
        pltpu.SemaphoreType.DMA,
        pltpu.SemaphoreType.DMA,
    ],
    compiler_params=pltpu.CompilerParams(needs_layout_passes=False),
)
def _sc_gather(idx_hbm, tableT_hbm, outT_hbm, idx_v, ring_v, colsT_v,
               sem_a, sem_b):
    wid = lax.axis_index("s") * _NC + lax.axis_index("c")
    base = wid * _BPW
    pltpu.sync_copy(idx_hbm.at[pl.ds(base, _BPW)], idx_v)
    lanes = lax.iota(jnp.int32, 16)

    def _fire(r, half, sem):
        vec = idx_v[pl.ds(r * _B, _B)]
        blk = lax.shift_right_logical(vec, 7)
        for j in range(_B):
            off = pl.multiple_of(blk[j] * 128, 128)
            pltpu.make_async_copy(
                tableT_hbm.at[:, pl.ds(off, 128)],
                ring_v.at[half + j],
                sem,
            ).start()

    def _drain_extract(r, half, sem):
        vec = idx_v[pl.ds(r * _B, _B)]
        blk = lax.shift_right_logical(vec, 7)
        col = lax.bitwise_and(vec, 127)
        for j in range(_B):
            off = pl.multiple_of(blk[j] * 128, 128)
            pltpu.make_async_copy(
                tableT_hbm.at[:, pl.ds(off, 128)],
                ring_v.at[half + j],
                sem,
            ).wait()
        for j in range(_B):
            v = plsc.load_gather(
                ring_v,
                [jnp.full((16,), half + j, jnp.int32), lanes,
                 jnp.full((16,), col[j], jnp.int32)],
            )
            plsc.store_scatter(
                colsT_v,
                [lanes, jnp.full((16,), r * _B + j, jnp.int32)],
                v,
            )

    _fire(0, 0, sem_a)
    _fire(1, _B, sem_b)

    def _pair(i, _):
        r = 2 * i
        _drain_extract(r, 0, sem_a)
        _fire(r + 2, 0, sem_a)
        _drain_extract(r + 1, _B, sem_b)
        _fire(r + 3, _B, sem_b)
        return _

    lax.fori_loop(0, _NROUND // 2 - 1, _pair, 0)
    _drain_extract(_NROUND - 2, 0, sem_a)
    _drain_extract(_NROUND - 1, _B, sem_b)
    pltpu.sync_copy(colsT_v, outT_hbm.at[:, pl.ds(base, _BPW)])


def _tc_body(ufT_ref, ages_ref, w16_ref, wageT_ref, biasT_ref, gb_ref,
             outT_ref):
    a = ages_ref[...]
    n = jnp.float32(_BATCH)
    mean = jnp.sum(a) / n
    d = a - mean
    var = jnp.sum(d * d) / n
    gamma = gb_ref[0, 0]
    beta = gb_ref[0, 1]
    an = d * lax.rsqrt(var + _EPS) * gamma + beta
    an = jnp.maximum(an, 0.0)
    ufT = jnp.maximum(ufT_ref[...], 0.0)
    outT_ref[...] = (
        jnp.dot(w16_ref[...], ufT, preferred_element_type=jnp.float32)
        + an * wageT_ref[...]
        + biasT_ref[...]
    )


def kernel(user_ids, ages, emb_table, bn_gamma, bn_beta, W, b):
    out_dim = W.shape[0]
    w_pad = jnp.zeros((_EMB, W.shape[1]), W.dtype).at[:out_dim].set(W)
    w16 = w_pad[:, :_EMB]
    wageT = w_pad[:, _EMB].reshape(_EMB, 1)
    biasT = jnp.zeros((_EMB, 1), b.dtype).at[:out_dim, 0].set(b)
    gb = jnp.stack([bn_gamma[0], bn_beta[0]]).reshape(1, 2)

    ufT = _sc_gather(user_ids, emb_table.T)

    outT_pad = pl.pallas_call(
        _tc_body,
        out_shape=jax.ShapeDtypeStruct((_EMB, _BATCH), jnp.float32),
    )(ufT, ages.reshape(1, _BATCH), w16, wageT, biasT, gb)
    return outT_pad.T[:, :out_dim]


# speedup vs baseline: 6.2215x; 1.0522x over previous
import functools

import jax
import jax.numpy as jnp
from jax import lax
from jax.experimental import pallas as pl
from jax.experimental.pallas import tpu as pltpu
from jax.experimental.pallas import tpu_sc as plsc

_BATCH = 16384
_EMB = 16
_NC = 2
_NS = 16
_NW = _NC * _NS
_BPW = _BATCH // _NW
_EPS = 1e-5
_B = 16
_NROUND = _BPW // _B

_mesh = plsc.VectorSubcoreMesh(core_axis_name="c", subcore_axis_name="s")


@functools.partial(
    pl.kernel,
    out_type=jax.ShapeDtypeStruct((_EMB, _BATCH), jnp.float32),
    mesh=_mesh,
    scratch_types=[
        pltpu.VMEM((_BPW,), jnp.int32),
        pltpu.VMEM((3 * _B, _EMB, 128), jnp.float32),
        pltpu.VMEM((_EMB, _BPW), jnp.float32),
        pltpu.SemaphoreType.DMA,
        pltpu.SemaphoreType.DMA,
        pltpu.SemaphoreType.DMA,
    ],
    compiler_params=pltpu.CompilerParams(needs_layout_passes=False),
)
def _sc_gather(idx_hbm, tableT_hbm, outT_hbm, idx_v, ring_v, colsT_v,
               sem_a, sem_b, sem_c):
    wid = lax.axis_index("s") * _NC + lax.axis_index("c")
    base = wid * _BPW
    pltpu.sync_copy(idx_hbm.at[pl.ds(base, _BPW)], idx_v)
    lanes = lax.iota(jnp.int32, 16)
    sems = (sem_a, sem_b, sem_c)

    def _fire(r, sec):
        vec = idx_v[pl.ds(r * _B, _B)]
        blk = lax.shift_right_logical(vec, 7)
        for j in range(_B):
            off = pl.multiple_of(blk[j] * 128, 128)
            pltpu.make_async_copy(
                tableT_hbm.at[:, pl.ds(off, 128)],
                ring_v.at[sec * _B + j],
                sems[sec],
            ).start()

    def _drain_extract(r, sec):
        vec = idx_v[pl.ds(r * _B, _B)]
        blk = lax.shift_right_logical(vec, 7)
        col = lax.bitwise_and(vec, 127)
        for j in range(_B):
            off = pl.multiple_of(blk[j] * 128, 128)
            pltpu.make_async_copy(
                tableT_hbm.at[:, pl.ds(off, 128)],
                ring_v.at[sec * _B + j],
                sems[sec],
            ).wait()
        for j in range(_B):
            v = plsc.load_gather(
                ring_v,
                [jnp.full((16,), sec * _B + j, jnp.int32), lanes,
                 jnp.full((16,), col[j], jnp.int32)],
            )
            plsc.store_scatter(
                colsT_v,
                [lanes, jnp.full((16,), r * _B + j, jnp.int32)],
                v,
            )

    _fire(0, 0)
    _fire(1, 1)
    _fire(2, 2)

    def _triple(i, _):
        r = 3 * i
        _drain_extract(r, 0)
        _fire(r + 3, 0)
        _drain_extract(r + 1, 1)
        _fire(r + 4, 1)
        _drain_extract(r + 2, 2)
        _fire(r + 5, 2)
        return _

    lax.fori_loop(0, _NROUND // 3 - 1, _triple, 0)
    _drain_extract(27, 0)
    _fire(30, 0)
    _drain_extract(28, 1)
    _fire(31, 1)
    _drain_extract(29, 2)
    _drain_extract(30, 0)
    _drain_extract(31, 1)
    pltpu.sync_copy(colsT_v, outT_hbm.at[:, pl.ds(base, _BPW)])


def _tc_body(ufT_ref, ages_ref, w_ref, b_ref, g_ref, bt_ref, outT_ref):
    a = ages_ref[...]
    n = jnp.float32(_BATCH)
    mean = jnp.sum(a) / n
    d = a - mean
    var = jnp.sum(d * d) / n
    an = d * lax.rsqrt(var + _EPS) * g_ref[0, 0] + bt_ref[0, 0]
    an = jnp.maximum(an, 0.0)
    ufT = jnp.maximum(ufT_ref[...], 0.0)
    w = w_ref[...]
    outT_ref[...] = (
        jnp.dot(w[:, :_EMB], ufT, preferred_element_type=jnp.float32)
        + an * w[:, _EMB:]
        + b_ref[...]
    )


def kernel(user_ids, ages, emb_table, bn_gamma, bn_beta, W, b):
    out_dim = W.shape[0]
    ufT = _sc_gather(user_ids, emb_table.T)
    outT = pl.pallas_call(
        _tc_body,
        out_shape=jax.ShapeDtypeStruct((out_dim, _BATCH), jnp.float32),
    )(ufT, ages.reshape(1, _BATCH), W, b.reshape(out_dim, 1),
      bn_gamma.reshape(1, 1), bn_beta.reshape(1, 1))
    return outT.T
